# Initial kernel scaffold; baseline (speedup 1.0000x reference)
import functools

import jax
import jax.numpy as jnp
from jax import lax
from jax.experimental import pallas as pl
from jax.experimental.pallas import tpu as pltpu
from jax.experimental.pallas import tpu_sc as plsc

N = 10000
F_IN = 128
HID = 64
E = 320000
NE = E + N
R = 2 * N + 32
W_COL = 72
CHUNK = 512
SUB = 128
NSUB = CHUNK // SUB
NPAD = N + 16


def _prep_body(x_ref, w_ref, a_ref, hp_ref, hn_ref, sa_ref, sb_ref):
    h = jnp.dot(x_ref[...], w_ref[...], preferred_element_type=jnp.float32)
    av = a_ref[...]
    dn = (((1,), (1,)), ((), ()))
    sa = lax.dot_general(h, av[:, :HID], dn, preferred_element_type=jnp.float32)
    sb = lax.dot_general(h, av[:, HID:], dn, preferred_element_type=jnp.float32)
    bp = jnp.exp(-sb)
    bn = jnp.exp(-0.2 * sb)
    z7 = jnp.zeros((h.shape[0], W_COL - HID - 1), jnp.float32)
    hp_ref[...] = jnp.concatenate([h * bp, bp, z7], axis=1)
    hn_ref[...] = jnp.concatenate([h * bn, bn, z7], axis=1)
    sa_ref[...] = sa
    sb_ref[...] = sb


def _combine_body(apply_elu, nc, acc_ref, sa_ref, out_ref):
    acc_p = acc_ref[0, 0:N, :]
    acc_m = acc_ref[0, N:2 * N, :]
    for i in range(1, nc):
        acc_p = acc_p + acc_ref[i, 0:N, :]
        acc_m = acc_m + acc_ref[i, N:2 * N, :]
    sa = sa_ref[...]
    ap = jnp.exp(-sa)
    an = jnp.exp(-0.2 * sa)
    num = ap * acc_p[:, 0:HID] + an * acc_m[:, 0:HID]
    den = ap * acc_p[:, HID:HID + 1] + an * acc_m[:, HID:HID + 1] + 1e-16
    o = num / den
    if apply_elu:
        o = jnp.where(o > 0.0, o, jnp.expm1(o))
    out_ref[...] = o


def _make_edge_kernel(nc, ns, epw):
    nchunk = epw // CHUNK
    rows_per_tile = R // ns
    zrows = rows_per_tile // 4
    mesh = plsc.VectorSubcoreMesh(core_axis_name="c", subcore_axis_name="s",
                                  num_cores=nc, num_subcores=ns)

    @functools.partial(
        pl.kernel,
        out_type=jax.ShapeDtypeStruct((nc, R, W_COL), jnp.float32),
        mesh=mesh,
        scratch_types=[
            pltpu.VMEM((NPAD,), jnp.float32),
            pltpu.VMEM((NPAD,), jnp.float32),
            pltpu.VMEM((CHUNK,), jnp.int32),
            pltpu.VMEM((CHUNK,), jnp.int32),
            pltpu.VMEM((NSUB, SUB), jnp.int32),
            pltpu.VMEM((NSUB, SUB), jnp.int32),
            pltpu.VMEM((NSUB, SUB, W_COL), jnp.float32),
            pltpu.VMEM((R // 16 // 4, W_COL), jnp.float32),
            pltpu.VMEM_SHARED((R, W_COL), jnp.float32),
            pltpu.SemaphoreType.DMA,
        ],
    )
    def edge_kernel(row_hbm, col_hbm, sa_hbm, sb_hbm, htab_hbm, zeros_hbm,
                    acc_out, sa_v, sb_v, row_v, col_v, gcol_v, grow_v,
                    rows_v, zbuf, acc_sh, sem):
        cid = lax.axis_index("c")
        sid = lax.axis_index("s")
        wid = sid * nc + cid
        pltpu.sync_copy(sa_hbm, sa_v)
        pltpu.sync_copy(sb_hbm, sb_v)
        pltpu.sync_copy(zeros_hbm, zbuf)
        base_row = sid * rows_per_tile
        for k in range(rows_per_tile // zrows):
            pltpu.sync_copy(zbuf, acc_sh.at[pl.ds(base_row + k * zrows, zrows)])
        plsc.subcore_barrier()

        def chunk_body(g, carry):
            base = wid * epw + g * CHUNK
            pltpu.sync_copy(row_hbm.at[pl.ds(base, CHUNK)], row_v)
            pltpu.sync_copy(col_hbm.at[pl.ds(base, CHUNK)], col_v)
            for q in range(NSUB):
                for j in range(SUB // 16):
                    o = q * SUB + j * 16
                    r = row_v[pl.ds(o, 16)]
                    c = col_v[pl.ds(o, 16)]
                    sa = plsc.load_gather(sa_v, [r])
                    sb = plsc.load_gather(sb_v, [c])
                    neg = (sa + sb) <= 0.0
                    offs = jnp.where(neg, N, 0).astype(jnp.int32)
                    gcol_v[q, pl.ds(j * 16, 16)] = c + offs
                    grow_v[q, pl.ds(j * 16, 16)] = r + offs
            copies = [
                pltpu.async_copy(htab_hbm.at[gcol_v.at[q]], rows_v.at[q], sem)
                for q in range(NSUB)
            ]
            for q in range(NSUB):
                copies[q].wait()
            for q in range(NSUB):
                pltpu.sync_copy(rows_v.at[q], acc_sh.at[grow_v.at[q]], add=True)
            return carry

        lax.fori_loop(0, nchunk, chunk_body, 0)
        plsc.subcore_barrier()
        for k in range(rows_per_tile // zrows):
            r0 = base_row + k * zrows
            pltpu.sync_copy(acc_sh.at[pl.ds(r0, zrows)], zbuf)
            pltpu.sync_copy(zbuf, acc_out.at[cid].at[pl.ds(r0, zrows)])

    return edge_kernel, rows_per_tile // 4


def kernel(x, adj_edge_index, W0, a0, W1, a1, W_out, a_out):
    info = plsc.get_sparse_core_info()
    nc, ns = info.num_cores, info.num_subcores
    nw = nc * ns
    epw = -(-NE // (nw * CHUNK)) * CHUNK
    epad = epw * nw

    edge_call, zrows = _make_edge_kernel(nc, ns, epw)

    prep_call = pl.pallas_call(
        _prep_body,
        out_shape=[
            jax.ShapeDtypeStruct((N, W_COL), jnp.float32),
            jax.ShapeDtypeStruct((N, W_COL), jnp.float32),
            jax.ShapeDtypeStruct((N, 1), jnp.float32),
            jax.ShapeDtypeStruct((N, 1), jnp.float32),
        ],
    )

    def combine_call(acc, sa, apply_elu):
        return pl.pallas_call(
            functools.partial(_combine_body, apply_elu, nc),
            out_shape=jax.ShapeDtypeStruct((N, HID), jnp.float32),
        )(acc, sa)

    loop = jnp.arange(N, dtype=jnp.int32)
    padi = jnp.full((epad - NE,), N, jnp.int32)
    row = jnp.concatenate([adj_edge_index[0], loop, padi])
    col = jnp.concatenate([adj_edge_index[1], loop, padi])
    zeros_blk = jnp.zeros((zrows, W_COL), jnp.float32)
    spad = jnp.zeros((NPAD - N,), jnp.float32)
    tpad = jnp.zeros((R - 2 * N, W_COL), jnp.float32)

    def layer(h_in, W, a, apply_elu):
        hp, hn, sa, sb = prep_call(h_in, W, a.reshape(1, -1))
        htab = jnp.concatenate([hp, hn, tpad], axis=0)
        sa_p = jnp.concatenate([sa[:, 0], spad])
        sb_p = jnp.concatenate([sb[:, 0], spad])
        acc = edge_call(row, col, sa_p, sb_p, htab, zeros_blk)
        return combine_call(acc, sa, apply_elu)

    h0 = layer(x, W0, a0, True)
    h1 = layer(x, W1, a1, True)
    hcat = jnp.concatenate([h0, h1], axis=1)
    return layer(hcat, W_out, a_out, False)


# speedup vs baseline: 6.5264x; 6.5264x over previous
import functools

import jax
import jax.numpy as jnp
from jax import lax
from jax.experimental import pallas as pl
from jax.experimental.pallas import tpu as pltpu
from jax.experimental.pallas import tpu_sc as plsc

N = 10000
F_IN = 128
HID = 64
E = 320000
NE = E + N
R = 2 * N + 480
W_COL = 72
CHUNK = 256
SUB = 128
NSUB = CHUNK // SUB
NPAD = N + 16


def _prep_body(x_ref, w_ref, a_ref, hp_ref, hn_ref, sab_ref, sa_ref):
    h = jnp.dot(x_ref[...], w_ref[...], preferred_element_type=jnp.float32)
    av = a_ref[...]
    dn = (((1,), (1,)), ((), ()))
    sa = lax.dot_general(h, av[:, :HID], dn, preferred_element_type=jnp.float32)
    sb = lax.dot_general(h, av[:, HID:], dn, preferred_element_type=jnp.float32)
    bp = jnp.exp(-sb)
    bn = jnp.exp(-0.2 * sb)
    z7 = jnp.zeros((h.shape[0], W_COL - HID - 1), jnp.float32)
    hp_ref[...] = jnp.concatenate([h * bp, bp, z7], axis=1)
    hn_ref[...] = jnp.concatenate([h * bn, bn, z7], axis=1)
    ua = lax.bitcast_convert_type(sa, jnp.int32)
    ub = lax.bitcast_convert_type(sb, jnp.int32)
    sab_ref[...] = (ua & jnp.int32(-65536)) | lax.shift_right_logical(
        ub, jnp.full_like(ub, 16))
    sa_ref[...] = sa


def _combine_body(apply_elu, nc, accp_ref, accm_ref, sa_ref, out_ref):
    acc_p = accp_ref[0]
    acc_m = accm_ref[0]
    for i in range(1, nc):
        acc_p = acc_p + accp_ref[i]
        acc_m = acc_m + accm_ref[i]
    sa = sa_ref[...]
    ap = jnp.exp(-sa)
    an = jnp.exp(-0.2 * sa)
    num = ap * acc_p[:, 0:HID] + an * acc_m[:, 0:HID]
    den = ap * acc_p[:, HID:HID + 1] + an * acc_m[:, HID:HID + 1] + 1e-16
    o = num / den
    if apply_elu:
        o = jnp.where(o > 0.0, o, jnp.exp(o) - 1.0)
    out_ref[...] = o


def _make_edge_kernel(nc, ns, epw):
    nchunk = epw // CHUNK
    rows_per_tile = R // ns
    assert rows_per_tile % 8 == 0
    mesh = plsc.VectorSubcoreMesh(core_axis_name="c", subcore_axis_name="s",
                                  num_cores=nc, num_subcores=ns)

    @functools.partial(
        pl.kernel,
        out_type=jax.ShapeDtypeStruct((nc, R, W_COL), jnp.float32),
        mesh=mesh,
        scratch_types=[
            pltpu.VMEM((NPAD,), jnp.int32),
            pltpu.VMEM((CHUNK,), jnp.int32),
            pltpu.VMEM((CHUNK,), jnp.int32),
            pltpu.VMEM((NSUB, SUB), jnp.int32),
            pltpu.VMEM((NSUB, SUB), jnp.int32),
            pltpu.VMEM((NSUB, SUB, W_COL), jnp.float32),
            pltpu.VMEM_SHARED((R, W_COL), jnp.float32),
            pltpu.SemaphoreType.DMA,
        ],
        compiler_params=pltpu.CompilerParams(needs_layout_passes=False,
                                             use_tc_tiling_on_sc=False),
    )
    def edge_kernel(row_hbm, col_hbm, sab_hbm, htab_hbm, zeros_hbm,
                    acc_out, sab_v, row_v, col_v, gcol_v, grow_v,
                    rows_v, acc_sh, sem):
        cid = lax.axis_index("c")
        sid = lax.axis_index("s")
        wid = sid * nc + cid
        pltpu.sync_copy(sab_hbm, sab_v)
        base_row = sid * rows_per_tile
        pltpu.sync_copy(zeros_hbm, acc_sh.at[pl.ds(base_row, rows_per_tile)])
        plsc.subcore_barrier()

        def chunk_body(g, carry):
            base = wid * epw + g * CHUNK
            pltpu.sync_copy(row_hbm.at[pl.ds(base, CHUNK)], row_v)
            pltpu.sync_copy(col_hbm.at[pl.ds(base, CHUNK)], col_v)
            for q in range(NSUB):
                for j in range(SUB // 16):
                    o = q * SUB + j * 16
                    r = row_v[pl.ds(o, 16)]
                    c = col_v[pl.ds(o, 16)]
                    wa = plsc.load_gather(sab_v, [r])
                    wb = plsc.load_gather(sab_v, [c])
                    sa = plsc.bitcast(wa & jnp.int32(-65536), jnp.float32)
                    sb = plsc.bitcast(
                        lax.shift_left(wb, jnp.full_like(wb, 16)), jnp.float32)
                    neg = (sa + sb) <= 0.0
                    offs = jnp.where(neg, N, 0).astype(jnp.int32)
                    gcol_v[q, pl.ds(j * 16, 16)] = c + offs
                    grow_v[q, pl.ds(j * 16, 16)] = r + offs
            copies = [
                pltpu.async_copy(htab_hbm.at[gcol_v.at[q]], rows_v.at[q], sem)
                for q in range(NSUB)
            ]
            for q in range(NSUB):
                copies[q].wait()
            for q in range(NSUB):
                pltpu.sync_copy(rows_v.at[q], acc_sh.at[grow_v.at[q]], add=True)
            return carry

        lax.fori_loop(0, nchunk, chunk_body, 0)
        plsc.subcore_barrier()
        pltpu.sync_copy(acc_sh.at[pl.ds(base_row, rows_per_tile)],
                        acc_out.at[cid].at[pl.ds(base_row, rows_per_tile)])

    return edge_kernel, rows_per_tile


def kernel(x, adj_edge_index, W0, a0, W1, a1, W_out, a_out):
    info = plsc.get_sparse_core_info()
    nc, ns = info.num_cores, info.num_subcores
    nw = nc * ns
    epw = -(-NE // (nw * CHUNK)) * CHUNK
    epad = epw * nw

    edge_call, zrows = _make_edge_kernel(nc, ns, epw)

    prep_call = pl.pallas_call(
        _prep_body,
        out_shape=[
            jax.ShapeDtypeStruct((N, W_COL), jnp.float32),
            jax.ShapeDtypeStruct((N, W_COL), jnp.float32),
            jax.ShapeDtypeStruct((N, 1), jnp.int32),
            jax.ShapeDtypeStruct((N, 1), jnp.float32),
        ],
    )

    cb = 2000
    assert N % cb == 0

    def combine_call(acc, sa, apply_elu):
        return pl.pallas_call(
            functools.partial(_combine_body, apply_elu, nc),
            grid=(N // cb,),
            in_specs=[
                pl.BlockSpec((nc, cb, W_COL), lambda i: (0, i, 0)),
                pl.BlockSpec((nc, cb, W_COL), lambda i: (0, N // cb + i, 0)),
                pl.BlockSpec((cb, 1), lambda i: (i, 0)),
            ],
            out_specs=pl.BlockSpec((cb, HID), lambda i: (i, 0)),
            out_shape=jax.ShapeDtypeStruct((N, HID), jnp.float32),
        )(acc, acc, sa)

    loop = jnp.arange(N, dtype=jnp.int32)
    padi = jnp.full((epad - NE,), N, jnp.int32)
    row = jnp.concatenate([adj_edge_index[0], loop, padi])
    col = jnp.concatenate([adj_edge_index[1], loop, padi])
    zeros_blk = jnp.zeros((zrows, W_COL), jnp.float32)
    spad = jnp.zeros((NPAD - N,), jnp.int32)
    tpad = jnp.zeros((R - 2 * N, W_COL), jnp.float32)

    def layer(h_in, W, a, apply_elu):
        hp, hn, sab, sa = prep_call(h_in, W, a.reshape(1, -1))
        htab = jnp.concatenate([hp, hn, tpad], axis=0)
        sab_p = jnp.concatenate([sab[:, 0], spad])
        acc = edge_call(row, col, sab_p, htab, zeros_blk)
        return combine_call(acc, sa, apply_elu)

    h0 = layer(x, W0, a0, True)
    h1 = layer(x, W1, a1, True)
    hcat = jnp.concatenate([h0, h1], axis=1)
    return layer(hcat, W_out, a_out, False)


# speedup vs baseline: 7.5903x; 1.1630x over previous
import functools

import jax
import jax.numpy as jnp
from jax import lax
from jax.experimental import pallas as pl
from jax.experimental.pallas import tpu as pltpu
from jax.experimental.pallas import tpu_sc as plsc

N = 10000
F_IN = 128
HID = 64
E = 320000
NE = E + N
R = 2 * N + 480
W_COL = 72
CHUNK = 128
NPAD = N + 16


def _prep_body(x_ref, w_ref, a_ref, hp_ref, hn_ref, sab_ref, sa_ref):
    h = jnp.dot(x_ref[...], w_ref[...], preferred_element_type=jnp.float32)
    av = a_ref[...]
    dn = (((1,), (1,)), ((), ()))
    sa = lax.dot_general(h, av[:, :HID], dn, preferred_element_type=jnp.float32)
    sb = lax.dot_general(h, av[:, HID:], dn, preferred_element_type=jnp.float32)
    bp = jnp.exp(-sb)
    bn = jnp.exp(-0.2 * sb)
    z7 = jnp.zeros((h.shape[0], W_COL - HID - 1), jnp.float32)
    hp_ref[...] = jnp.concatenate([h * bp, bp, z7], axis=1)
    hn_ref[...] = jnp.concatenate([h * bn, bn, z7], axis=1)
    ua = lax.bitcast_convert_type(sa, jnp.int32)
    ub = lax.bitcast_convert_type(sb, jnp.int32)
    sab_ref[...] = (ua & jnp.int32(-65536)) | lax.shift_right_logical(
        ub, jnp.full_like(ub, 16))
    sa_ref[...] = sa


def _combine_body(apply_elu, nc, accp_ref, accm_ref, sa_ref, out_ref):
    acc_p = accp_ref[0]
    acc_m = accm_ref[0]
    for i in range(1, nc):
        acc_p = acc_p + accp_ref[i]
        acc_m = acc_m + accm_ref[i]
    sa = sa_ref[...]
    ap = jnp.exp(-sa)
    an = jnp.exp(-0.2 * sa)
    num = ap * acc_p[:, 0:HID] + an * acc_m[:, 0:HID]
    den = ap * acc_p[:, HID:HID + 1] + an * acc_m[:, HID:HID + 1] + 1e-16
    o = num / den
    if apply_elu:
        o = jnp.where(o > 0.0, o, jnp.exp(o) - 1.0)
    out_ref[...] = o


def _make_edge_kernel(nc, ns, epw):
    nchunk = epw // CHUNK
    assert nchunk % 2 == 0
    rows_per_tile = R // ns
    assert rows_per_tile % 8 == 0
    mesh = plsc.VectorSubcoreMesh(core_axis_name="c", subcore_axis_name="s",
                                  num_cores=nc, num_subcores=ns)

    @functools.partial(
        pl.kernel,
        out_type=jax.ShapeDtypeStruct((nc, R, W_COL), jnp.float32),
        mesh=mesh,
        scratch_types=[
            pltpu.VMEM((NPAD,), jnp.int32),
            pltpu.VMEM((2, CHUNK), jnp.int32),
            pltpu.VMEM((2, CHUNK), jnp.int32),
            pltpu.VMEM((2, CHUNK), jnp.int32),
            pltpu.VMEM((2, CHUNK), jnp.int32),
            pltpu.VMEM((2, CHUNK, W_COL), jnp.float32),
            pltpu.VMEM_SHARED((R, W_COL), jnp.float32),
            pltpu.SemaphoreType.DMA,
            pltpu.SemaphoreType.DMA,
            pltpu.SemaphoreType.DMA,
            pltpu.SemaphoreType.DMA,
        ],
        compiler_params=pltpu.CompilerParams(needs_layout_passes=False,
                                             use_tc_tiling_on_sc=False),
    )
    def edge_kernel(row_hbm, col_hbm, sab_hbm, htab_hbm, zeros_hbm,
                    acc_out, sab_v, row_v, col_v, gcol_v, grow_v,
                    rows_v, acc_sh, rc0, rc1, ga0, ga1):
        rc = (rc0, rc1)
        ga = (ga0, ga1)
        cid = lax.axis_index("c")
        sid = lax.axis_index("s")
        wid = sid * nc + cid
        ebase = wid * epw

        def fire_rowcol(b, g):
            pltpu.async_copy(row_hbm.at[pl.ds(ebase + g * CHUNK, CHUNK)],
                             row_v.at[b], rc[b])
            pltpu.async_copy(col_hbm.at[pl.ds(ebase + g * CHUNK, CHUNK)],
                             col_v.at[b], rc[b])

        def wait_rowcol(b):
            pltpu.make_async_copy(row_hbm.at[pl.ds(0, CHUNK)], row_v.at[b],
                                  rc[b]).wait()
            pltpu.make_async_copy(col_hbm.at[pl.ds(0, CHUNK)], col_v.at[b],
                                  rc[b]).wait()

        def compute_idx(b):
            for j in range(CHUNK // 16):
                o = j * 16
                r = row_v[b, pl.ds(o, 16)]
                c = col_v[b, pl.ds(o, 16)]
                wa = plsc.load_gather(sab_v, [r])
                wb = plsc.load_gather(sab_v, [c])
                sa = plsc.bitcast(wa & jnp.int32(-65536), jnp.float32)
                sb = plsc.bitcast(
                    lax.shift_left(wb, jnp.full_like(wb, 16)), jnp.float32)
                neg = (sa + sb) <= 0.0
                offs = jnp.where(neg, N, 0).astype(jnp.int32)
                gcol_v[b, pl.ds(o, 16)] = c + offs
                grow_v[b, pl.ds(o, 16)] = r + offs

        def fire_gather(b):
            return pltpu.async_copy(htab_hbm.at[gcol_v.at[b]], rows_v.at[b],
                                    ga[b])

        pltpu.sync_copy(sab_hbm, sab_v)
        fire_rowcol(0, 0)
        fire_rowcol(1, 1)
        base_row = sid * rows_per_tile
        pltpu.sync_copy(zeros_hbm, acc_sh.at[pl.ds(base_row, rows_per_tile)])
        plsc.subcore_barrier()

        def pipe_body(i, carry):
            wait_rowcol(0)
            compute_idx(0)
            fire_rowcol(0, 2 * i + 2)
            d0 = fire_gather(0)
            wait_rowcol(1)
            compute_idx(1)
            fire_rowcol(1, 2 * i + 3)
            d1 = fire_gather(1)
            d0.wait()
            pltpu.sync_copy(rows_v.at[0], acc_sh.at[grow_v.at[0]], add=True)
            d1.wait()
            pltpu.sync_copy(rows_v.at[1], acc_sh.at[grow_v.at[1]], add=True)
            return carry

        lax.fori_loop(0, nchunk // 2, pipe_body, 0)
        wait_rowcol(0)
        wait_rowcol(1)
        plsc.subcore_barrier()
        pltpu.sync_copy(acc_sh.at[pl.ds(base_row, rows_per_tile)],
                        acc_out.at[cid].at[pl.ds(base_row, rows_per_tile)])

    return edge_kernel, rows_per_tile


def kernel(x, adj_edge_index, W0, a0, W1, a1, W_out, a_out):
    info = plsc.get_sparse_core_info()
    nc, ns = info.num_cores, info.num_subcores
    nw = nc * ns
    epw = -(-NE // (nw * 2 * CHUNK)) * 2 * CHUNK
    epad = epw * nw

    edge_call, zrows = _make_edge_kernel(nc, ns, epw)

    prep_call = pl.pallas_call(
        _prep_body,
        out_shape=[
            jax.ShapeDtypeStruct((N, W_COL), jnp.float32),
            jax.ShapeDtypeStruct((N, W_COL), jnp.float32),
            jax.ShapeDtypeStruct((N, 1), jnp.int32),
            jax.ShapeDtypeStruct((N, 1), jnp.float32),
        ],
    )

    cb = 2000
    assert N % cb == 0

    def combine_call(acc, sa, apply_elu):
        return pl.pallas_call(
            functools.partial(_combine_body, apply_elu, nc),
            grid=(N // cb,),
            in_specs=[
                pl.BlockSpec((nc, cb, W_COL), lambda i: (0, i, 0)),
                pl.BlockSpec((nc, cb, W_COL), lambda i: (0, N // cb + i, 0)),
                pl.BlockSpec((cb, 1), lambda i: (i, 0)),
            ],
            out_specs=pl.BlockSpec((cb, HID), lambda i: (i, 0)),
            out_shape=jax.ShapeDtypeStruct((N, HID), jnp.float32),
        )(acc, acc, sa)

    loop = jnp.arange(N, dtype=jnp.int32)
    padi = jnp.full((epad - NE + 2 * CHUNK,), N, jnp.int32)
    row = jnp.concatenate([adj_edge_index[0], loop, padi])
    col = jnp.concatenate([adj_edge_index[1], loop, padi])
    zeros_blk = jnp.zeros((zrows, W_COL), jnp.float32)
    spad = jnp.zeros((NPAD - N,), jnp.int32)
    tpad = jnp.zeros((R - 2 * N, W_COL), jnp.float32)

    def layer(h_in, W, a, apply_elu):
        hp, hn, sab, sa = prep_call(h_in, W, a.reshape(1, -1))
        htab = jnp.concatenate([hp, hn, tpad], axis=0)
        sab_p = jnp.concatenate([sab[:, 0], spad])
        acc = edge_call(row, col, sab_p, htab, zeros_blk)
        return combine_call(acc, sa, apply_elu)

    h0 = layer(x, W0, a0, True)
    h1 = layer(x, W1, a1, True)
    hcat = jnp.concatenate([h0, h1], axis=1)
    return layer(hcat, W_out, a_out, False)


# speedup vs baseline: 12.0763x; 1.5910x over previous
import functools

import jax
import jax.numpy as jnp
from jax import lax
from jax.experimental import pallas as pl
from jax.experimental.pallas import tpu as pltpu
from jax.experimental.pallas import tpu_sc as plsc

N = 10000
F_IN = 128
HID = 64
E = 320000
NE = E + N
R = 2 * N + 480
W_COL = 72
CHUNK = 128
NPAD = N + 16


def _prep_body(x_ref, w_ref, a_ref, hp_ref, hn_ref, sab_ref, sa_ref):
    h = jnp.dot(x_ref[...], w_ref[...], preferred_element_type=jnp.float32)
    av = a_ref[...]
    dn = (((1,), (1,)), ((), ()))
    sa = lax.dot_general(h, av[:, :HID], dn, preferred_element_type=jnp.float32)
    sb = lax.dot_general(h, av[:, HID:], dn, preferred_element_type=jnp.float32)
    bp = jnp.exp(-sb)
    bn = jnp.exp(-0.2 * sb)
    z7 = jnp.zeros((h.shape[0], W_COL - HID - 1), jnp.float32)
    hp_ref[...] = jnp.concatenate([h * bp, bp, z7], axis=1)
    hn_ref[...] = jnp.concatenate([h * bn, bn, z7], axis=1)
    ua = lax.bitcast_convert_type(sa, jnp.int32)
    ub = lax.bitcast_convert_type(sb, jnp.int32)
    sab_ref[...] = (ua & jnp.int32(-65536)) | lax.shift_right_logical(
        ub, jnp.full_like(ub, 16))
    sa_ref[...] = sa


def _combine_body(apply_elu, nc, accp_ref, accm_ref, sa_ref, out_ref):
    acc_p = accp_ref[0]
    acc_m = accm_ref[0]
    for i in range(1, nc):
        acc_p = acc_p + accp_ref[i]
        acc_m = acc_m + accm_ref[i]
    sa = sa_ref[...]
    ap = jnp.exp(-sa)
    an = jnp.exp(-0.2 * sa)
    num = ap * acc_p[:, 0:HID] + an * acc_m[:, 0:HID]
    den = ap * acc_p[:, HID:HID + 1] + an * acc_m[:, HID:HID + 1] + 1e-16
    o = num / den
    if apply_elu:
        o = jnp.where(o > 0.0, o, jnp.exp(o) - 1.0)
    out_ref[...] = o


def _make_edge_kernel(nc, ns, pa, pb):
    assert nc == 2
    rows_per_tile = R // ns
    assert rows_per_tile % 8 == 0
    mesh = plsc.VectorSubcoreMesh(core_axis_name="c", subcore_axis_name="s",
                                  num_cores=nc, num_subcores=ns)

    @functools.partial(
        pl.kernel,
        out_type=jax.ShapeDtypeStruct((nc, R, W_COL), jnp.float32),
        mesh=mesh,
        scratch_types=[
            pltpu.VMEM((NPAD,), jnp.int32),
            pltpu.VMEM((2, CHUNK), jnp.int32),
            pltpu.VMEM((2, CHUNK), jnp.int32),
            pltpu.VMEM((2, CHUNK), jnp.int32),
            pltpu.VMEM((2, CHUNK), jnp.int32),
            pltpu.VMEM((2, CHUNK, W_COL), jnp.float32),
            pltpu.VMEM_SHARED((R, W_COL), jnp.float32),
            pltpu.SemaphoreType.DMA,
            pltpu.SemaphoreType.DMA,
            pltpu.SemaphoreType.DMA,
            pltpu.SemaphoreType.DMA,
        ],
        compiler_params=pltpu.CompilerParams(needs_layout_passes=False,
                                             use_tc_tiling_on_sc=False),
    )
    def edge_kernel(row_hbm, col_hbm, sab_hbm, htab_hbm, zeros_hbm,
                    acc_out, sab_v, row_v, col_v, gcol_v, grow_v,
                    rows_v, acc_sh, rc0, rc1, ga0, ga1):
        rc = (rc0, rc1)
        ga = (ga0, ga1)
        cid = lax.axis_index("c")
        sid = lax.axis_index("s")
        is0 = cid == 0
        npairs = lax.select(is0, jnp.int32(pa), jnp.int32(pb))
        ebase = lax.select(is0, sid * (pa * 2 * CHUNK),
                           pa * ns * 2 * CHUNK + sid * (pb * 2 * CHUNK))

        def fire_rowcol(b, g):
            pltpu.async_copy(row_hbm.at[pl.ds(ebase + g * CHUNK, CHUNK)],
                             row_v.at[b], rc[b])
            pltpu.async_copy(col_hbm.at[pl.ds(ebase + g * CHUNK, CHUNK)],
                             col_v.at[b], rc[b])

        def wait_rowcol(b):
            pltpu.make_async_copy(row_hbm.at[pl.ds(0, CHUNK)], row_v.at[b],
                                  rc[b]).wait()
            pltpu.make_async_copy(col_hbm.at[pl.ds(0, CHUNK)], col_v.at[b],
                                  rc[b]).wait()

        def compute_idx(b):
            for j in range(CHUNK // 16):
                o = j * 16
                r = row_v[b, pl.ds(o, 16)]
                c = col_v[b, pl.ds(o, 16)]
                wa = plsc.load_gather(sab_v, [r])
                wb = plsc.load_gather(sab_v, [c])
                sa = plsc.bitcast(wa & jnp.int32(-65536), jnp.float32)
                sb = plsc.bitcast(
                    lax.shift_left(wb, jnp.full_like(wb, 16)), jnp.float32)
                neg = (sa + sb) <= 0.0
                offs = jnp.where(neg, N, 0).astype(jnp.int32)
                gcol_v[b, pl.ds(o, 16)] = c + offs
                grow_v[b, pl.ds(o, 16)] = r + offs

        def fire_gather(b):
            return pltpu.async_copy(htab_hbm.at[gcol_v.at[b]], rows_v.at[b],
                                    ga[b])

        pltpu.sync_copy(sab_hbm, sab_v)
        fire_rowcol(0, 0)
        fire_rowcol(1, 1)
        base_row = sid * rows_per_tile
        pltpu.sync_copy(zeros_hbm, acc_sh.at[pl.ds(base_row, rows_per_tile)])
        plsc.subcore_barrier()

        def pipe_body(i, carry):
            wait_rowcol(0)
            compute_idx(0)
            fire_rowcol(0, 2 * i + 2)
            d0 = fire_gather(0)
            wait_rowcol(1)
            compute_idx(1)
            fire_rowcol(1, 2 * i + 3)
            d1 = fire_gather(1)
            d0.wait()
            pltpu.sync_copy(rows_v.at[0], acc_sh.at[grow_v.at[0]], add=True)
            d1.wait()
            pltpu.sync_copy(rows_v.at[1], acc_sh.at[grow_v.at[1]], add=True)
            return carry

        lax.fori_loop(0, npairs, pipe_body, 0)
        wait_rowcol(0)
        wait_rowcol(1)
        plsc.subcore_barrier()
        pltpu.sync_copy(acc_sh.at[pl.ds(base_row, rows_per_tile)],
                        acc_out.at[cid].at[pl.ds(base_row, rows_per_tile)])

    return edge_kernel, rows_per_tile


def kernel(x, adj_edge_index, W0, a0, W1, a1, W_out, a_out):
    info = plsc.get_sparse_core_info()
    nc, ns = info.num_cores, info.num_subcores
    pairs = -(-NE // (2 * CHUNK * ns))
    pa = int(round(pairs * 0.715))
    pb = pairs - pa
    epad = ns * (pa + pb) * 2 * CHUNK

    edge_call, zrows = _make_edge_kernel(nc, ns, pa, pb)

    prep_call = pl.pallas_call(
        _prep_body,
        out_shape=[
            jax.ShapeDtypeStruct((N, W_COL), jnp.float32),
            jax.ShapeDtypeStruct((N, W_COL), jnp.float32),
            jax.ShapeDtypeStruct((N, 1), jnp.int32),
            jax.ShapeDtypeStruct((N, 1), jnp.float32),
        ],
    )

    cb = 2000
    assert N % cb == 0

    def combine_call(acc, sa, apply_elu):
        return pl.pallas_call(
            functools.partial(_combine_body, apply_elu, nc),
            grid=(N // cb,),
            in_specs=[
                pl.BlockSpec((nc, cb, W_COL), lambda i: (0, i, 0)),
                pl.BlockSpec((nc, cb, W_COL), lambda i: (0, N // cb + i, 0)),
                pl.BlockSpec((cb, 1), lambda i: (i, 0)),
            ],
            out_specs=pl.BlockSpec((cb, HID), lambda i: (i, 0)),
            out_shape=jax.ShapeDtypeStruct((N, HID), jnp.float32),
        )(acc, acc, sa)

    loop = jnp.arange(N, dtype=jnp.int32)
    padi = jnp.full((epad - NE + 4 * CHUNK,), N, jnp.int32)
    row = jnp.concatenate([adj_edge_index[0], loop, padi])
    col = jnp.concatenate([adj_edge_index[1], loop, padi])
    zeros_blk = jnp.zeros((zrows, W_COL), jnp.float32)
    spad = jnp.zeros((NPAD - N,), jnp.int32)
    tpad = jnp.zeros((R - 2 * N, W_COL), jnp.float32)

    def layer(h_in, W, a, apply_elu):
        hp, hn, sab, sa = prep_call(h_in, W, a.reshape(1, -1))
        htab = jnp.concatenate([hp, hn, tpad], axis=0)
        sab_p = jnp.concatenate([sab[:, 0], spad])
        acc = edge_call(row, col, sab_p, htab, zeros_blk)
        return combine_call(acc, sa, apply_elu)

    h0 = layer(x, W0, a0, True)
    h1 = layer(x, W1, a1, True)
    hcat = jnp.concatenate([h0, h1], axis=1)
    return layer(hcat, W_out, a_out, False)


# speedup vs baseline: 13.3839x; 1.1083x over previous
import functools

import jax
import jax.numpy as jnp
from jax import lax
from jax.experimental import pallas as pl
from jax.experimental.pallas import tpu as pltpu
from jax.experimental.pallas import tpu_sc as plsc

N = 10000
F_IN = 128
HID = 64
E = 320000
NE = E + N
R = 2 * N + 480
W_COL = 72
CHUNK = 128
NPAD = N + 16


def _prep_body(x_ref, w_ref, a_ref, htab_ref, sab_ref, sa_ref):
    t = pl.program_id(0)
    h = jnp.dot(x_ref[...], w_ref[...], preferred_element_type=jnp.float32)
    av = a_ref[...]
    dn = (((1,), (1,)), ((), ()))
    sa = lax.dot_general(h, av[:, :HID], dn, preferred_element_type=jnp.float32)
    sb = lax.dot_general(h, av[:, HID:], dn, preferred_element_type=jnp.float32)
    kap = jnp.where(t == 0, 1.0, 0.2)
    b = jnp.exp(-kap * sb)
    z7 = jnp.zeros((h.shape[0], W_COL - HID - 1), jnp.float32)
    htab_ref[...] = jnp.concatenate([h * b, b, z7], axis=1)
    ua = lax.bitcast_convert_type(sa, jnp.int32)
    ub = lax.bitcast_convert_type(sb, jnp.int32)
    sab_ref[...] = (ua & jnp.int32(-65536)) | lax.shift_right_logical(
        ub, jnp.full_like(ub, 16))
    sa_ref[...] = sa


def _combine_body(apply_elu, nc, accp_ref, accm_ref, sa_ref, out_ref):
    acc_p = accp_ref[0]
    acc_m = accm_ref[0]
    for i in range(1, nc):
        acc_p = acc_p + accp_ref[i]
        acc_m = acc_m + accm_ref[i]
    sa = sa_ref[...]
    ap = jnp.exp(-sa)
    an = jnp.exp(-0.2 * sa)
    num = ap * acc_p[:, 0:HID] + an * acc_m[:, 0:HID]
    den = ap * acc_p[:, HID:HID + 1] + an * acc_m[:, HID:HID + 1] + 1e-16
    o = num / den
    if apply_elu:
        o = jnp.where(o > 0.0, o, jnp.exp(o) - 1.0)
    out_ref[...] = o


def _make_edge_kernel(nc, ns, pa, pb):
    assert nc == 2
    rows_per_tile = R // ns
    assert rows_per_tile % 8 == 0
    mesh = plsc.VectorSubcoreMesh(core_axis_name="c", subcore_axis_name="s",
                                  num_cores=nc, num_subcores=ns)

    @functools.partial(
        pl.kernel,
        out_type=jax.ShapeDtypeStruct((nc, R, W_COL), jnp.float32),
        mesh=mesh,
        scratch_types=[
            pltpu.VMEM((NPAD,), jnp.int32),
            pltpu.VMEM((2, CHUNK), jnp.int32),
            pltpu.VMEM((2, CHUNK), jnp.int32),
            pltpu.VMEM((2, CHUNK), jnp.int32),
            pltpu.VMEM((2, CHUNK), jnp.int32),
            pltpu.VMEM((2, CHUNK, W_COL), jnp.float32),
            pltpu.VMEM_SHARED((R, W_COL), jnp.float32),
            pltpu.SemaphoreType.DMA,
            pltpu.SemaphoreType.DMA,
            pltpu.SemaphoreType.DMA,
            pltpu.SemaphoreType.DMA,
        ],
        compiler_params=pltpu.CompilerParams(needs_layout_passes=False,
                                             use_tc_tiling_on_sc=False),
    )
    def edge_kernel(row_hbm, col_hbm, sab_hbm, htab_hbm, zeros_hbm,
                    acc_out, sab_v, row_v, col_v, gcol_v, grow_v,
                    rows_v, acc_sh, rc0, rc1, ga0, ga1):
        rc = (rc0, rc1)
        ga = (ga0, ga1)
        cid = lax.axis_index("c")
        sid = lax.axis_index("s")
        is0 = cid == 0
        npairs = lax.select(is0, jnp.int32(pa), jnp.int32(pb))
        ebase = lax.select(is0, sid * (pa * 2 * CHUNK),
                           pa * ns * 2 * CHUNK + sid * (pb * 2 * CHUNK))

        def fire_rowcol(b, g):
            pltpu.async_copy(row_hbm.at[pl.ds(ebase + g * CHUNK, CHUNK)],
                             row_v.at[b], rc[b])
            pltpu.async_copy(col_hbm.at[pl.ds(ebase + g * CHUNK, CHUNK)],
                             col_v.at[b], rc[b])

        def wait_rowcol(b):
            pltpu.make_async_copy(row_hbm.at[pl.ds(0, CHUNK)], row_v.at[b],
                                  rc[b]).wait()
            pltpu.make_async_copy(col_hbm.at[pl.ds(0, CHUNK)], col_v.at[b],
                                  rc[b]).wait()

        def compute_idx(b):
            for j in range(CHUNK // 16):
                o = j * 16
                r = row_v[b, pl.ds(o, 16)]
                c = col_v[b, pl.ds(o, 16)]
                wa = plsc.load_gather(sab_v, [r])
                wb = plsc.load_gather(sab_v, [c])
                sa = plsc.bitcast(wa & jnp.int32(-65536), jnp.float32)
                sb = plsc.bitcast(
                    lax.shift_left(wb, jnp.full_like(wb, 16)), jnp.float32)
                neg = (sa + sb) <= 0.0
                offs = jnp.where(neg, N, 0).astype(jnp.int32)
                gcol_v[b, pl.ds(o, 16)] = c + offs
                grow_v[b, pl.ds(o, 16)] = r + offs

        def fire_gather(b):
            return pltpu.async_copy(htab_hbm.at[gcol_v.at[b]], rows_v.at[b],
                                    ga[b])

        pltpu.sync_copy(sab_hbm, sab_v)
        fire_rowcol(0, 0)
        fire_rowcol(1, 1)
        base_row = sid * rows_per_tile
        pltpu.sync_copy(zeros_hbm, acc_sh.at[pl.ds(base_row, rows_per_tile)])
        plsc.subcore_barrier()

        def pipe_body(i, carry):
            wait_rowcol(0)
            compute_idx(0)
            fire_rowcol(0, 2 * i + 2)
            d0 = fire_gather(0)
            wait_rowcol(1)
            compute_idx(1)
            fire_rowcol(1, 2 * i + 3)
            d1 = fire_gather(1)
            d0.wait()
            s0 = pltpu.async_copy(rows_v.at[0], acc_sh.at[grow_v.at[0]],
                                  ga[0], add=True)
            d1.wait()
            s1 = pltpu.async_copy(rows_v.at[1], acc_sh.at[grow_v.at[1]],
                                  ga[1], add=True)
            s0.wait()
            s1.wait()
            return carry

        lax.fori_loop(0, npairs, pipe_body, 0)
        wait_rowcol(0)
        wait_rowcol(1)
        plsc.subcore_barrier()
        pltpu.sync_copy(acc_sh.at[pl.ds(base_row, rows_per_tile)],
                        acc_out.at[cid].at[pl.ds(base_row, rows_per_tile)])

    return edge_kernel, rows_per_tile


def kernel(x, adj_edge_index, W0, a0, W1, a1, W_out, a_out):
    info = plsc.get_sparse_core_info()
    nc, ns = info.num_cores, info.num_subcores
    pairs = -(-NE // (2 * CHUNK * ns))
    pa = int(round(pairs * 0.68))
    pb = pairs - pa
    epad = ns * (pa + pb) * 2 * CHUNK

    edge_call, zrows = _make_edge_kernel(nc, ns, pa, pb)

    prep_call = pl.pallas_call(
        _prep_body,
        grid=(2,),
        in_specs=[
            pl.BlockSpec((N, F_IN), lambda t: (0, 0)),
            pl.BlockSpec((F_IN, HID), lambda t: (0, 0)),
            pl.BlockSpec((1, 2 * HID), lambda t: (0, 0)),
        ],
        out_specs=[
            pl.BlockSpec((N, W_COL), lambda t: (t, 0)),
            pl.BlockSpec((N, 1), lambda t: (0, 0)),
            pl.BlockSpec((N, 1), lambda t: (0, 0)),
        ],
        out_shape=[
            jax.ShapeDtypeStruct((R, W_COL), jnp.float32),
            jax.ShapeDtypeStruct((N, 1), jnp.int32),
            jax.ShapeDtypeStruct((N, 1), jnp.float32),
        ],
    )

    cb = 2000
    assert N % cb == 0

    def combine_call(acc, sa, apply_elu):
        return pl.pallas_call(
            functools.partial(_combine_body, apply_elu, nc),
            grid=(N // cb,),
            in_specs=[
                pl.BlockSpec((nc, cb, W_COL), lambda i: (0, i, 0)),
                pl.BlockSpec((nc, cb, W_COL), lambda i: (0, N // cb + i, 0)),
                pl.BlockSpec((cb, 1), lambda i: (i, 0)),
            ],
            out_specs=pl.BlockSpec((cb, HID), lambda i: (i, 0)),
            out_shape=jax.ShapeDtypeStruct((N, HID), jnp.float32),
        )(acc, acc, sa)

    loop = jnp.arange(N, dtype=jnp.int32)
    padi = jnp.full((epad - NE + 4 * CHUNK,), N, jnp.int32)
    row = jnp.concatenate([adj_edge_index[0], loop, padi])
    col = jnp.concatenate([adj_edge_index[1], loop, padi])
    zeros_blk = jnp.zeros((zrows, W_COL), jnp.float32)
    spad = jnp.zeros((NPAD - N,), jnp.int32)

    def layer(h_in, W, a, apply_elu):
        htab, sab, sa = prep_call(h_in, W, a.reshape(1, -1))
        sab_p = jnp.concatenate([sab[:, 0], spad])
        acc = edge_call(row, col, sab_p, htab, zeros_blk)
        return combine_call(acc, sa, apply_elu)

    h0 = layer(x, W0, a0, True)
    h1 = layer(x, W1, a1, True)
    hcat = jnp.concatenate([h0, h1], axis=1)
    return layer(hcat, W_out, a_out, False)


# speedup vs baseline: 14.3750x; 1.0740x over previous
import functools

import jax
import jax.numpy as jnp
from jax import lax
from jax.experimental import pallas as pl
from jax.experimental.pallas import tpu as pltpu
from jax.experimental.pallas import tpu_sc as plsc

N = 10000
F_IN = 128
HID = 64
E = 320000
NE = E + N
NSLOT = 3
R = 2 * N + 224
W_COL = 72
CHUNK = 128
NPAD = N + 16


def _prep_body(x_ref, w_ref, a_ref, htab_ref, sab_ref, sa_ref):
    t = pl.program_id(0)
    h = jnp.dot(x_ref[...], w_ref[...], preferred_element_type=jnp.float32)
    av = a_ref[...]
    dn = (((1,), (1,)), ((), ()))
    sa = lax.dot_general(h, av[:, :HID], dn, preferred_element_type=jnp.float32)
    sb = lax.dot_general(h, av[:, HID:], dn, preferred_element_type=jnp.float32)
    kap = jnp.where(t == 0, 1.0, 0.2)
    b = jnp.exp(-kap * sb)
    z7 = jnp.zeros((h.shape[0], W_COL - HID - 1), jnp.float32)
    htab_ref[...] = jnp.concatenate([h * b, b, z7], axis=1)
    ua = lax.bitcast_convert_type(sa, jnp.int32)
    ub = lax.bitcast_convert_type(sb, jnp.int32)
    sab_ref[...] = (ua & jnp.int32(-65536)) | lax.shift_right_logical(
        ub, jnp.full_like(ub, 16))
    sa_ref[...] = sa


def _combine_body(apply_elu, nc, accp_ref, accm_ref, sa_ref, out_ref):
    acc_p = accp_ref[0]
    acc_m = accm_ref[0]
    for i in range(1, nc):
        acc_p = acc_p + accp_ref[i]
        acc_m = acc_m + accm_ref[i]
    sa = sa_ref[...]
    ap = jnp.exp(-sa)
    an = jnp.exp(-0.2 * sa)
    num = ap * acc_p[:, 0:HID] + an * acc_m[:, 0:HID]
    den = ap * acc_p[:, HID:HID + 1] + an * acc_m[:, HID:HID + 1] + 1e-16
    o = num / den
    if apply_elu:
        o = jnp.where(o > 0.0, o, jnp.exp(o) - 1.0)
    out_ref[...] = o


def _make_edge_kernel(nc, ns, pa, pb):
    assert nc == 2
    rows_per_tile = R // ns
    assert rows_per_tile % 8 == 0
    mesh = plsc.VectorSubcoreMesh(core_axis_name="c", subcore_axis_name="s",
                                  num_cores=nc, num_subcores=ns)

    @functools.partial(
        pl.kernel,
        out_type=jax.ShapeDtypeStruct((nc, R, W_COL), jnp.float32),
        mesh=mesh,
        scratch_types=[
            pltpu.VMEM((NPAD,), jnp.int32),
            pltpu.VMEM((NSLOT, CHUNK), jnp.int32),
            pltpu.VMEM((NSLOT, CHUNK), jnp.int32),
            pltpu.VMEM((NSLOT, CHUNK), jnp.int32),
            pltpu.VMEM((NSLOT, CHUNK), jnp.int32),
            pltpu.VMEM((NSLOT, CHUNK, W_COL), jnp.float32),
            pltpu.VMEM_SHARED((R, W_COL), jnp.float32),
        ] + [pltpu.SemaphoreType.DMA] * (2 * NSLOT),
        compiler_params=pltpu.CompilerParams(needs_layout_passes=False,
                                             use_tc_tiling_on_sc=False),
    )
    def edge_kernel(row_hbm, col_hbm, sab_hbm, htab_hbm, zeros_hbm,
                    acc_out, sab_v, row_v, col_v, gcol_v, grow_v,
                    rows_v, acc_sh, *sems):
        rc = sems[:NSLOT]
        ga = sems[NSLOT:]
        cid = lax.axis_index("c")
        sid = lax.axis_index("s")
        is0 = cid == 0
        npairs = lax.select(is0, jnp.int32(pa), jnp.int32(pb))
        ebase = lax.select(is0, sid * (pa * NSLOT * CHUNK),
                           pa * ns * NSLOT * CHUNK + sid * (pb * NSLOT * CHUNK))

        def fire_rowcol(b, g):
            pltpu.async_copy(row_hbm.at[pl.ds(ebase + g * CHUNK, CHUNK)],
                             row_v.at[b], rc[b])
            pltpu.async_copy(col_hbm.at[pl.ds(ebase + g * CHUNK, CHUNK)],
                             col_v.at[b], rc[b])

        def wait_rowcol(b):
            pltpu.make_async_copy(row_hbm.at[pl.ds(0, CHUNK)], row_v.at[b],
                                  rc[b]).wait()
            pltpu.make_async_copy(col_hbm.at[pl.ds(0, CHUNK)], col_v.at[b],
                                  rc[b]).wait()

        def compute_idx(b):
            for j in range(CHUNK // 16):
                o = j * 16
                r = row_v[b, pl.ds(o, 16)]
                c = col_v[b, pl.ds(o, 16)]
                wa = plsc.load_gather(sab_v, [r])
                wb = plsc.load_gather(sab_v, [c])
                sa = plsc.bitcast(wa & jnp.int32(-65536), jnp.float32)
                sb = plsc.bitcast(
                    lax.shift_left(wb, jnp.full_like(wb, 16)), jnp.float32)
                neg = (sa + sb) <= 0.0
                offs = jnp.where(neg, N, 0).astype(jnp.int32)
                gcol_v[b, pl.ds(o, 16)] = c + offs
                grow_v[b, pl.ds(o, 16)] = r + offs

        def fire_gather(b):
            return pltpu.async_copy(htab_hbm.at[gcol_v.at[b]], rows_v.at[b],
                                    ga[b])

        pltpu.sync_copy(sab_hbm, sab_v)
        for b in range(NSLOT):
            fire_rowcol(b, b)
        base_row = sid * rows_per_tile
        pltpu.sync_copy(zeros_hbm, acc_sh.at[pl.ds(base_row, rows_per_tile)])
        plsc.subcore_barrier()

        def pipe_body(i, carry):
            ds = []
            for b in range(NSLOT):
                wait_rowcol(b)
                compute_idx(b)
                fire_rowcol(b, NSLOT * i + NSLOT + b)
                ds.append(fire_gather(b))
            ss = []
            for b in range(NSLOT):
                ds[b].wait()
                ss.append(pltpu.async_copy(rows_v.at[b],
                                           acc_sh.at[grow_v.at[b]],
                                           ga[b], add=True))
            for s in ss:
                s.wait()
            return carry

        lax.fori_loop(0, npairs, pipe_body, 0)
        for b in range(NSLOT):
            wait_rowcol(b)
        plsc.subcore_barrier()
        pltpu.sync_copy(acc_sh.at[pl.ds(base_row, rows_per_tile)],
                        acc_out.at[cid].at[pl.ds(base_row, rows_per_tile)])

    return edge_kernel, rows_per_tile


def kernel(x, adj_edge_index, W0, a0, W1, a1, W_out, a_out):
    info = plsc.get_sparse_core_info()
    nc, ns = info.num_cores, info.num_subcores
    pairs = -(-NE // (NSLOT * CHUNK * ns))
    pa = int(round(pairs * 0.68))
    pb = pairs - pa
    epad = ns * (pa + pb) * NSLOT * CHUNK

    edge_call, zrows = _make_edge_kernel(nc, ns, pa, pb)

    prep_call = pl.pallas_call(
        _prep_body,
        grid=(2,),
        in_specs=[
            pl.BlockSpec((N, F_IN), lambda t: (0, 0)),
            pl.BlockSpec((F_IN, HID), lambda t: (0, 0)),
            pl.BlockSpec((1, 2 * HID), lambda t: (0, 0)),
        ],
        out_specs=[
            pl.BlockSpec((N, W_COL), lambda t: (t, 0)),
            pl.BlockSpec((N, 1), lambda t: (0, 0)),
            pl.BlockSpec((N, 1), lambda t: (0, 0)),
        ],
        out_shape=[
            jax.ShapeDtypeStruct((R, W_COL), jnp.float32),
            jax.ShapeDtypeStruct((N, 1), jnp.int32),
            jax.ShapeDtypeStruct((N, 1), jnp.float32),
        ],
    )

    cb = 2000
    assert N % cb == 0

    def combine_call(acc, sa, apply_elu):
        return pl.pallas_call(
            functools.partial(_combine_body, apply_elu, nc),
            grid=(N // cb,),
            in_specs=[
                pl.BlockSpec((nc, cb, W_COL), lambda i: (0, i, 0)),
                pl.BlockSpec((nc, cb, W_COL), lambda i: (0, N // cb + i, 0)),
                pl.BlockSpec((cb, 1), lambda i: (i, 0)),
            ],
            out_specs=pl.BlockSpec((cb, HID), lambda i: (i, 0)),
            out_shape=jax.ShapeDtypeStruct((N, HID), jnp.float32),
        )(acc, acc, sa)

    loop = jnp.arange(N, dtype=jnp.int32)
    padi = jnp.full((epad - NE + 4 * CHUNK,), N, jnp.int32)
    row = jnp.concatenate([adj_edge_index[0], loop, padi])
    col = jnp.concatenate([adj_edge_index[1], loop, padi])
    zeros_blk = jnp.zeros((zrows, W_COL), jnp.float32)
    spad = jnp.zeros((NPAD - N,), jnp.int32)

    def layer(h_in, W, a, apply_elu):
        htab, sab, sa = prep_call(h_in, W, a.reshape(1, -1))
        sab_p = jnp.concatenate([sab[:, 0], spad])
        acc = edge_call(row, col, sab_p, htab, zeros_blk)
        return combine_call(acc, sa, apply_elu)

    h0 = layer(x, W0, a0, True)
    h1 = layer(x, W1, a1, True)
    hcat = jnp.concatenate([h0, h1], axis=1)
    return layer(hcat, W_out, a_out, False)


# speedup vs baseline: 15.8982x; 1.1060x over previous
import functools

import jax
import jax.numpy as jnp
from jax import lax
from jax.experimental import pallas as pl
from jax.experimental.pallas import tpu as pltpu
from jax.experimental.pallas import tpu_sc as plsc

N = 10000
F_IN = 128
HID = 64
E = 320000
NE = E + N
NSLOT = 3
R = 2 * N + 224
W_COL = 72
CHUNK = 128
NPAD = N + 16


def _prep_body(x_ref, w_ref, a_ref, htab_ref, sab_ref, sa_ref):
    t = pl.program_id(0)
    h = jnp.dot(x_ref[...], w_ref[...], preferred_element_type=jnp.float32)
    av = a_ref[...]
    dn = (((1,), (1,)), ((), ()))
    sa = lax.dot_general(h, av[:, :HID], dn, preferred_element_type=jnp.float32)
    sb = lax.dot_general(h, av[:, HID:], dn, preferred_element_type=jnp.float32)
    kap = jnp.where(t == 0, 1.0, 0.2)
    b = jnp.exp(-kap * sb)
    z7 = jnp.zeros((h.shape[0], W_COL - HID - 1), jnp.float32)
    htab_ref[...] = jnp.concatenate([h * b, b, z7], axis=1)
    ua = lax.bitcast_convert_type(sa, jnp.int32)
    ub = lax.bitcast_convert_type(sb, jnp.int32)
    sab_ref[...] = (ua & jnp.int32(-65536)) | lax.shift_right_logical(
        ub, jnp.full_like(ub, 16))
    sa_ref[...] = sa


def _combine_body(apply_elu, nc, accp_ref, accm_ref, sa_ref, out_ref):
    acc_p = accp_ref[0]
    acc_m = accm_ref[0]
    for i in range(1, nc):
        acc_p = acc_p + accp_ref[i]
        acc_m = acc_m + accm_ref[i]
    sa = sa_ref[...]
    ap = jnp.exp(-sa)
    an = jnp.exp(-0.2 * sa)
    num = ap * acc_p[:, 0:HID] + an * acc_m[:, 0:HID]
    den = ap * acc_p[:, HID:HID + 1] + an * acc_m[:, HID:HID + 1] + 1e-16
    o = num / den
    if apply_elu:
        o = jnp.where(o > 0.0, o, jnp.exp(o) - 1.0)
    out_ref[...] = o


def _make_edge_kernel(nc, ns, pa, pb):
    assert nc == 2
    rows_per_tile = R // ns
    assert rows_per_tile % 8 == 0
    mesh = plsc.VectorSubcoreMesh(core_axis_name="c", subcore_axis_name="s",
                                  num_cores=nc, num_subcores=ns)

    @functools.partial(
        pl.kernel,
        out_type=jax.ShapeDtypeStruct((nc, R, 128), jnp.float32),
        mesh=mesh,
        scratch_types=[
            pltpu.VMEM((NPAD,), jnp.int32),
            pltpu.VMEM((NSLOT, CHUNK), jnp.int32),
            pltpu.VMEM((NSLOT, CHUNK), jnp.int32),
            pltpu.VMEM((NSLOT, CHUNK), jnp.int32),
            pltpu.VMEM((NSLOT, CHUNK), jnp.int32),
            pltpu.VMEM((NSLOT, CHUNK, W_COL), jnp.float32),
            pltpu.VMEM_SHARED((R, W_COL), jnp.float32),
        ] + [pltpu.SemaphoreType.DMA] * (2 * NSLOT),
        compiler_params=pltpu.CompilerParams(needs_layout_passes=False,
                                             use_tc_tiling_on_sc=False),
    )
    def edge_kernel(row_hbm, col_hbm, sab_hbm, htab_hbm, zeros_hbm,
                    acc_out, sab_v, row_v, col_v, gcol_v, grow_v,
                    rows_v, acc_sh, *sems):
        rc = sems[:NSLOT]
        ga = sems[NSLOT:]
        cid = lax.axis_index("c")
        sid = lax.axis_index("s")
        is0 = cid == 0
        npairs = lax.select(is0, jnp.int32(pa), jnp.int32(pb))
        ebase = lax.select(is0, sid * (pa * NSLOT * CHUNK),
                           pa * ns * NSLOT * CHUNK + sid * (pb * NSLOT * CHUNK))

        def fire_rowcol(b, g):
            pltpu.async_copy(row_hbm.at[pl.ds(ebase + g * CHUNK, CHUNK)],
                             row_v.at[b], rc[b])
            pltpu.async_copy(col_hbm.at[pl.ds(ebase + g * CHUNK, CHUNK)],
                             col_v.at[b], rc[b])

        def wait_rowcol(b):
            pltpu.make_async_copy(row_hbm.at[pl.ds(0, CHUNK)], row_v.at[b],
                                  rc[b]).wait()
            pltpu.make_async_copy(col_hbm.at[pl.ds(0, CHUNK)], col_v.at[b],
                                  rc[b]).wait()

        def compute_idx(b):
            for j in range(CHUNK // 16):
                o = j * 16
                r = row_v[b, pl.ds(o, 16)]
                c = col_v[b, pl.ds(o, 16)]
                wa = plsc.load_gather(sab_v, [r])
                wb = plsc.load_gather(sab_v, [c])
                sa = plsc.bitcast(wa & jnp.int32(-65536), jnp.float32)
                sb = plsc.bitcast(
                    lax.shift_left(wb, jnp.full_like(wb, 16)), jnp.float32)
                neg = (sa + sb) <= 0.0
                offs = jnp.where(neg, N, 0).astype(jnp.int32)
                gcol_v[b, pl.ds(o, 16)] = c + offs
                grow_v[b, pl.ds(o, 16)] = r + offs

        def fire_gather(b):
            return pltpu.async_copy(htab_hbm.at[gcol_v.at[b]], rows_v.at[b],
                                    ga[b])

        pltpu.sync_copy(sab_hbm, sab_v)
        for b in range(NSLOT):
            fire_rowcol(b, b)
        base_row = sid * rows_per_tile
        pltpu.sync_copy(zeros_hbm, acc_sh.at[pl.ds(base_row, rows_per_tile)])
        plsc.subcore_barrier()

        def pipe_body(i, carry):
            ds = []
            for b in range(NSLOT):
                wait_rowcol(b)
                compute_idx(b)
                fire_rowcol(b, NSLOT * i + NSLOT + b)
                ds.append(fire_gather(b))
            ss = []
            for b in range(NSLOT):
                ds[b].wait()
                ss.append(pltpu.async_copy(rows_v.at[b],
                                           acc_sh.at[grow_v.at[b]],
                                           ga[b], add=True))
            for s in ss:
                s.wait()
            return carry

        lax.fori_loop(0, npairs, pipe_body, 0)
        for b in range(NSLOT):
            wait_rowcol(b)
        plsc.subcore_barrier()
        pltpu.sync_copy(
            acc_sh.at[pl.ds(base_row, rows_per_tile)],
            acc_out.at[cid].at[pl.ds(base_row, rows_per_tile), pl.ds(0, W_COL)])

    return edge_kernel, rows_per_tile


def kernel(x, adj_edge_index, W0, a0, W1, a1, W_out, a_out):
    info = plsc.get_sparse_core_info()
    nc, ns = info.num_cores, info.num_subcores
    pairs = -(-NE // (NSLOT * CHUNK * ns))
    pa = int(round(pairs * 0.68))
    pb = pairs - pa
    epad = ns * (pa + pb) * NSLOT * CHUNK

    edge_call, zrows = _make_edge_kernel(nc, ns, pa, pb)

    prep_call = pl.pallas_call(
        _prep_body,
        grid=(2,),
        in_specs=[
            pl.BlockSpec((N, F_IN), lambda t: (0, 0)),
            pl.BlockSpec((F_IN, HID), lambda t: (0, 0)),
            pl.BlockSpec((1, 2 * HID), lambda t: (0, 0)),
        ],
        out_specs=[
            pl.BlockSpec((N, W_COL), lambda t: (t, 0)),
            pl.BlockSpec((N, 1), lambda t: (0, 0)),
            pl.BlockSpec((N, 1), lambda t: (0, 0)),
        ],
        out_shape=[
            jax.ShapeDtypeStruct((R, W_COL), jnp.float32),
            jax.ShapeDtypeStruct((N, 1), jnp.int32),
            jax.ShapeDtypeStruct((N, 1), jnp.float32),
        ],
    )

    cb = 2000
    assert N % cb == 0

    def combine_call(acc, sa, apply_elu):
        return pl.pallas_call(
            functools.partial(_combine_body, apply_elu, nc),
            grid=(N // cb,),
            in_specs=[
                pl.BlockSpec((nc, cb, 128), lambda i: (0, i, 0)),
                pl.BlockSpec((nc, cb, 128), lambda i: (0, N // cb + i, 0)),
                pl.BlockSpec((cb, 1), lambda i: (i, 0)),
            ],
            out_specs=pl.BlockSpec((cb, HID), lambda i: (i, 0)),
            out_shape=jax.ShapeDtypeStruct((N, HID), jnp.float32),
        )(acc, acc, sa)

    loop = jnp.arange(N, dtype=jnp.int32)
    padi = jnp.full((epad - NE + 4 * CHUNK,), N, jnp.int32)
    row = jnp.concatenate([adj_edge_index[0], loop, padi])
    col = jnp.concatenate([adj_edge_index[1], loop, padi])
    zeros_blk = jnp.zeros((zrows, W_COL), jnp.float32)
    spad = jnp.zeros((NPAD - N,), jnp.int32)

    def layer(h_in, W, a, apply_elu):
        htab, sab, sa = prep_call(h_in, W, a.reshape(1, -1))
        sab_p = jnp.concatenate([sab[:, 0], spad])
        acc = edge_call(row, col, sab_p, htab, zeros_blk)
        return combine_call(acc, sa, apply_elu)

    h0 = layer(x, W0, a0, True)
    h1 = layer(x, W1, a1, True)
    hcat = jnp.concatenate([h0, h1], axis=1)
    return layer(hcat, W_out, a_out, False)
